# baseline (speedup 1.0000x reference)
import functools

import jax
import jax.numpy as jnp
from jax import lax
from jax.experimental import pallas as pl
from jax.experimental.pallas import tpu as pltpu
from jax.experimental.pallas import tpu_sc as plsc

B, P, F = 1024, 200, 8
D = 64
W = 2 * F + 1 + F + 2 + D
NC, NS = 2, 16
NW = NC * NS
BPW = B // NW
GH = P // 2


def _sc_body(y_hbm, x_hbm, sh_hbm, bc_hbm, idx_hbm, tab_hbm, out_hbm,
             asm_v, emb_v, idx_v, st_v, sem):
    wid = lax.axis_index("s") * NC + lax.axis_index("c")

    def batch_body(j, carry):
        b = wid * BPW + j
        pltpu.sync_copy(idx_hbm.at[b], idx_v)
        pltpu.sync_copy(bc_hbm.at[b], st_v)
        g0 = pltpu.async_copy(tab_hbm.at[idx_v.at[0]],
                              emb_v.at[pl.ds(0, GH)], sem)
        g1 = pltpu.async_copy(tab_hbm.at[idx_v.at[1]],
                              emb_v.at[pl.ds(GH, GH)], sem)
        pat = st_v[...]

        def row_body(r, c):
            asm_v[r, pl.ds(2 * F + 1, 16)] = pat
            return c

        lax.fori_loop(0, P, row_body, 0, unroll=8)
        pltpu.sync_copy(y_hbm.at[b], asm_v.at[:, pl.ds(0, F)])
        pltpu.sync_copy(x_hbm.at[b], asm_v.at[:, pl.ds(F, F)])
        pltpu.sync_copy(sh_hbm.at[b], asm_v.at[:, pl.ds(2 * F, 1)])
        g0.wait()
        g1.wait()

        def emb_body(r, c):
            for k in range(D // 16):
                asm_v[r, pl.ds(W - D + 16 * k, 16)] = emb_v[r, pl.ds(16 * k, 16)]
            return c

        lax.fori_loop(0, P, emb_body, 0, unroll=4)
        pltpu.sync_copy(asm_v, out_hbm.at[b])
        return carry

    lax.fori_loop(0, BPW, batch_body, 0)


_sc_call = pl.kernel(
    _sc_body,
    out_type=jax.ShapeDtypeStruct((B, P, W), jnp.float32),
    mesh=plsc.VectorSubcoreMesh(core_axis_name="c", subcore_axis_name="s"),
    scratch_types=[
        pltpu.VMEM((P, W), jnp.float32),
        pltpu.VMEM((P, D), jnp.float32),
        pltpu.VMEM((2, GH), jnp.int32),
        pltpu.VMEM((16,), jnp.float32),
        pltpu.SemaphoreType.DMA,
    ],
    compiler_params=pltpu.CompilerParams(use_tc_tiling_on_sc=False),
)


def kernel(pv_y_osgb_fourier, pv_x_osgb_fourier, pv_system_row_number, pv_x_osgb,
           pv_surface_height, pv_time_utc_fourier, solar_azimuth, solar_elevation,
           embedding_table, start_idx=0):
    t = 12 + start_idx
    time_row = jnp.take(pv_time_utc_fourier, t, axis=1)
    az = jnp.take(solar_azimuth, t, axis=1)[:, None]
    el = jnp.take(solar_elevation, t, axis=1)[:, None]
    bcast = jnp.concatenate(
        [time_row, az, el, jnp.zeros((B, 16 - F - 2), jnp.float32)], axis=1)
    idx = pv_system_row_number.astype(jnp.int32).reshape(B, 2, GH)
    sh = pv_surface_height[..., None]
    return _sc_call(pv_y_osgb_fourier, pv_x_osgb_fourier, sh, bcast, idx,
                    embedding_table)


# speedup vs baseline: 2.9819x; 2.9819x over previous
import jax
import jax.numpy as jnp
from jax import lax
from jax.experimental import pallas as pl
from jax.experimental.pallas import tpu as pltpu
from jax.experimental.pallas import tpu_sc as plsc

B, P, F = 1024, 200, 8
V, D = 100000, 64
W = 2 * F + 1 + F + 2 + D
NC, NS = 2, 16
NW = NC * NS
PB = P // 8


def _sc_body(y_hbm, x_hbm, sh_hbm, bc_hbm, idx_hbm, tab_hbm, out_hbm,
             row_v, idx_v, os_v, st_v, bcr_v):
    wid = lax.axis_index("s") * NC + lax.axis_index("c")

    @pl.when(wid < PB)
    def _dense():
        p0 = pl.multiple_of(wid * 8, 8)
        pltpu.sync_copy(sh_hbm.at[pl.ds(p0, 8), :], st_v)
        pltpu.sync_copy(st_v, out_hbm.at[2 * F, pl.ds(p0, 8), :])
        for f in range(F):
            pltpu.sync_copy(y_hbm.at[pl.ds(p0, 8), f, :], st_v)
            pltpu.sync_copy(st_v, out_hbm.at[f, pl.ds(p0, 8), :])
            pltpu.sync_copy(x_hbm.at[pl.ds(p0, 8), f, :], st_v)
            pltpu.sync_copy(st_v, out_hbm.at[F + f, pl.ds(p0, 8), :])

    @pl.when(wid >= NW - 10)
    def _bcast():
        j = wid - (NW - 10)
        pltpu.sync_copy(bc_hbm.at[j, :], bcr_v)

        def rep_body(c, carry):
            c16 = pl.multiple_of(c * 16, 16)
            pat = bcr_v[pl.ds(c16, 16)]
            for r in range(8):
                st_v[r, pl.ds(c16, 16)] = pat
            return carry

        lax.fori_loop(0, B // 16, rep_body, 0, unroll=4)

        def wr_body(pb, carry):
            p0 = pl.multiple_of(pb * 8, 8)
            pltpu.sync_copy(st_v, out_hbm.at[2 * F + 1 + j, pl.ds(p0, 8), :])
            return carry

        lax.fori_loop(0, PB, wr_body, 0)

    for t in range(2):
        k = wid * 2 + t
        pltpu.sync_copy(tab_hbm.at[k, :], row_v)

        def pb_body(pb, carry):
            p0 = pl.multiple_of(pb * 8, 8)
            pltpu.sync_copy(idx_hbm.at[pl.ds(p0, 8), :], idx_v)

            def g_body(c, carry2):
                c16 = pl.multiple_of(c * 16, 16)
                for r in range(8):
                    iv = idx_v[r, pl.ds(c16, 16)]
                    os_v[r, pl.ds(c16, 16)] = plsc.load_gather(row_v, [iv])
                return carry2

            lax.fori_loop(0, B // 16, g_body, 0, unroll=2)
            pltpu.sync_copy(os_v, out_hbm.at[W - D + k, pl.ds(p0, 8), :])
            return carry

        lax.fori_loop(0, PB, pb_body, 0)


_sc_call = pl.kernel(
    _sc_body,
    out_type=jax.ShapeDtypeStruct((W, P, B), jnp.float32),
    mesh=plsc.VectorSubcoreMesh(core_axis_name="c", subcore_axis_name="s"),
    scratch_types=[
        pltpu.VMEM((V,), jnp.float32),
        pltpu.VMEM((8, B), jnp.int32),
        pltpu.VMEM((8, B), jnp.float32),
        pltpu.VMEM((8, B), jnp.float32),
        pltpu.VMEM((B,), jnp.float32),
    ],
    compiler_params=pltpu.CompilerParams(needs_layout_passes=False),
)


def kernel(pv_y_osgb_fourier, pv_x_osgb_fourier, pv_system_row_number, pv_x_osgb,
           pv_surface_height, pv_time_utc_fourier, solar_azimuth, solar_elevation,
           embedding_table, start_idx=0):
    t = 12 + start_idx
    y_t = jnp.transpose(pv_y_osgb_fourier, (1, 2, 0))
    x_t = jnp.transpose(pv_x_osgb_fourier, (1, 2, 0))
    sh_t = jnp.transpose(pv_surface_height, (1, 0))
    idx_t = jnp.transpose(pv_system_row_number.astype(jnp.int32), (1, 0))
    tab_t = jnp.transpose(embedding_table, (1, 0))
    time_t = jnp.transpose(pv_time_utc_fourier, (1, 2, 0))
    time_sl = lax.dynamic_index_in_dim(time_t, t, 0, keepdims=False)
    az_sl = lax.dynamic_index_in_dim(jnp.transpose(solar_azimuth, (1, 0)),
                                     t, 0, keepdims=True)
    el_sl = lax.dynamic_index_in_dim(jnp.transpose(solar_elevation, (1, 0)),
                                     t, 0, keepdims=True)
    bc = jnp.concatenate(
        [time_sl, az_sl, el_sl, jnp.zeros((16 - F - 2, B), jnp.float32)], axis=0)
    out_t = _sc_call(y_t, x_t, sh_t, bc, idx_t, tab_t)
    return jnp.transpose(out_t, (2, 1, 0))


# speedup vs baseline: 5.9708x; 2.0024x over previous
import jax
import jax.numpy as jnp
from jax import lax
from jax.experimental import pallas as pl
from jax.experimental.pallas import tpu as pltpu
from jax.experimental.pallas import tpu_sc as plsc

B, P, F = 1024, 200, 8
V, D = 100000, 64
W = 2 * F + 1 + F + 2 + D
NC, NS = 2, 16
NW = NC * NS
PB = P // 8
H = B // 2


def _sc_body(y_hbm, x_hbm, sh_hbm, bc_hbm, idx_hbm, tab_hbm, out_hbm,
             row_v, ia_v, ib_v, oa_v, ob_v, sa_v, sb_v, bcr_v,
             sem_ia, sem_ib, sem_oa, sem_ob, sem_da, sem_db):
    wid = lax.axis_index("s") * NC + lax.axis_index("c")

    @pl.when(wid < PB)
    def _dense():
        p0 = pl.multiple_of(wid * 8, 8)

        def src(t):
            kind, f, b0 = t
            if kind == 0:
                return y_hbm.at[pl.ds(p0, 8), f, pl.ds(b0, H)]
            if kind == 1:
                return x_hbm.at[pl.ds(p0, 8), f, pl.ds(b0, H)]
            return sh_hbm.at[pl.ds(p0, 8), pl.ds(b0, H)]

        def dst(t):
            kind, f, b0 = t
            fo = f if kind == 0 else (F + f if kind == 1 else 2 * F)
            return out_hbm.at[fo, pl.ds(p0, 8), pl.ds(b0, H)]

        transfers = [(kind, f, b0)
                     for kind in (0, 1) for f in range(F) for b0 in (0, H)]
        transfers += [(2, 0, 0), (2, 0, H)]
        bufs = [(sa_v, sem_da), (sb_v, sem_db)]
        handles = [None, None]
        handles[0] = pltpu.async_copy(src(transfers[0]), sa_v, sem_da)
        for t, tr in enumerate(transfers):
            s = t % 2
            if t + 1 < len(transfers):
                nb, nsem = bufs[(t + 1) % 2]
                handles[(t + 1) % 2] = pltpu.async_copy(
                    src(transfers[t + 1]), nb, nsem)
            handles[s].wait()
            pltpu.sync_copy(bufs[s][0], dst(tr))

    @pl.when(wid >= NW - 10)
    def _bcast():
        j = wid - (NW - 10)
        fo = 2 * F + 1 + j
        pltpu.sync_copy(bc_hbm.at[j, :], bcr_v)

        for half, st in ((0, sa_v), (H, sb_v)):
            def rep_body(c, carry, half=half, st=st):
                cl = pl.multiple_of(c * 16, 16)
                cg = pl.multiple_of(half + c * 16, 16)
                pat = bcr_v[pl.ds(cg, 16)]
                for r in range(8):
                    st[r, pl.ds(cl, 16)] = pat
                return carry

            lax.fori_loop(0, H // 16, rep_body, 0, unroll=4)

        def fire_body(pb, carry):
            p0 = pl.multiple_of(pb * 8, 8)
            pltpu.async_copy(sa_v, out_hbm.at[fo, pl.ds(p0, 8), pl.ds(0, H)],
                             sem_da)
            pltpu.async_copy(sb_v, out_hbm.at[fo, pl.ds(p0, 8), pl.ds(H, H)],
                             sem_db)
            return carry

        lax.fori_loop(0, PB, fire_body, 0)

        def drain_body(pb, carry):
            p0 = pl.multiple_of(pb * 8, 8)
            pltpu.make_async_copy(
                sa_v, out_hbm.at[fo, pl.ds(p0, 8), pl.ds(0, H)], sem_da).wait()
            pltpu.make_async_copy(
                sb_v, out_hbm.at[fo, pl.ds(p0, 8), pl.ds(H, H)], sem_db).wait()
            return carry

        lax.fori_loop(0, PB, drain_body, 0)

    def gather(idx_buf, os_buf):
        def g_body(c, carry):
            c16 = pl.multiple_of(c * 16, 16)
            ivs = [idx_buf[r, pl.ds(c16, 16)] for r in range(8)]
            gs = [plsc.load_gather(row_v, [iv]) for iv in ivs]
            for r in range(8):
                os_buf[r, pl.ds(c16, 16)] = gs[r]
            return carry

        lax.fori_loop(0, H // 16, g_body, 0, unroll=2)

    def idx_src(p0, b0):
        return idx_hbm.at[pl.ds(p0, 8), pl.ds(b0, H)]

    for t in range(2):
        k = wid * 2 + t
        fo = W - D + k
        pltpu.sync_copy(tab_hbm.at[k, :], row_v)

        def out_dst(p0, b0, fo=fo):
            return out_hbm.at[fo, pl.ds(p0, 8), pl.ds(b0, H)]

        pltpu.async_copy(idx_src(0, 0), ia_v, sem_ia)
        pltpu.async_copy(idx_src(0, H), ib_v, sem_ib)
        pltpu.make_async_copy(idx_src(0, 0), ia_v, sem_ia).wait()
        gather(ia_v, oa_v)
        pltpu.async_copy(oa_v, out_dst(0, 0), sem_oa)
        pltpu.async_copy(idx_src(8, 0), ia_v, sem_ia)
        pltpu.make_async_copy(idx_src(0, H), ib_v, sem_ib).wait()
        gather(ib_v, ob_v)
        pltpu.async_copy(ob_v, out_dst(0, H), sem_ob)
        pltpu.async_copy(idx_src(8, H), ib_v, sem_ib)

        def row_body(i, carry):
            p0 = pl.multiple_of(i * 8, 8)
            pn = pl.multiple_of(i * 8 + 8, 8)
            pltpu.make_async_copy(idx_src(p0, 0), ia_v, sem_ia).wait()
            pltpu.make_async_copy(oa_v, out_dst(p0, 0), sem_oa).wait()
            gather(ia_v, oa_v)
            pltpu.async_copy(oa_v, out_dst(p0, 0), sem_oa)
            pltpu.async_copy(idx_src(pn, 0), ia_v, sem_ia)
            pltpu.make_async_copy(idx_src(p0, H), ib_v, sem_ib).wait()
            pltpu.make_async_copy(ob_v, out_dst(p0, H), sem_ob).wait()
            gather(ib_v, ob_v)
            pltpu.async_copy(ob_v, out_dst(p0, H), sem_ob)
            pltpu.async_copy(idx_src(pn, H), ib_v, sem_ib)
            return carry

        lax.fori_loop(1, PB - 1, row_body, 0)

        pl_last = pl.multiple_of((PB - 1) * 8, 8)
        pltpu.make_async_copy(idx_src(pl_last, 0), ia_v, sem_ia).wait()
        pltpu.make_async_copy(oa_v, out_dst(pl_last, 0), sem_oa).wait()
        gather(ia_v, oa_v)
        pltpu.async_copy(oa_v, out_dst(pl_last, 0), sem_oa)
        pltpu.make_async_copy(idx_src(pl_last, H), ib_v, sem_ib).wait()
        pltpu.make_async_copy(ob_v, out_dst(pl_last, H), sem_ob).wait()
        gather(ib_v, ob_v)
        pltpu.async_copy(ob_v, out_dst(pl_last, H), sem_ob)
        pltpu.make_async_copy(oa_v, out_dst(pl_last, 0), sem_oa).wait()
        pltpu.make_async_copy(ob_v, out_dst(pl_last, H), sem_ob).wait()


_sc_call = pl.kernel(
    _sc_body,
    out_type=jax.ShapeDtypeStruct((W, P, B), jnp.float32),
    mesh=plsc.VectorSubcoreMesh(core_axis_name="c", subcore_axis_name="s"),
    scratch_types=[
        pltpu.VMEM((V,), jnp.float32),
        pltpu.VMEM((8, H), jnp.int32),
        pltpu.VMEM((8, H), jnp.int32),
        pltpu.VMEM((8, H), jnp.float32),
        pltpu.VMEM((8, H), jnp.float32),
        pltpu.VMEM((8, H), jnp.float32),
        pltpu.VMEM((8, H), jnp.float32),
        pltpu.VMEM((B,), jnp.float32),
        pltpu.SemaphoreType.DMA,
        pltpu.SemaphoreType.DMA,
        pltpu.SemaphoreType.DMA,
        pltpu.SemaphoreType.DMA,
        pltpu.SemaphoreType.DMA,
        pltpu.SemaphoreType.DMA,
    ],
    compiler_params=pltpu.CompilerParams(needs_layout_passes=False),
)


def kernel(pv_y_osgb_fourier, pv_x_osgb_fourier, pv_system_row_number, pv_x_osgb,
           pv_surface_height, pv_time_utc_fourier, solar_azimuth, solar_elevation,
           embedding_table, start_idx=0):
    t = 12 + start_idx
    y_t = jnp.transpose(pv_y_osgb_fourier, (1, 2, 0))
    x_t = jnp.transpose(pv_x_osgb_fourier, (1, 2, 0))
    sh_t = jnp.transpose(pv_surface_height, (1, 0))
    idx_t = jnp.transpose(pv_system_row_number.astype(jnp.int32), (1, 0))
    tab_t = jnp.transpose(embedding_table, (1, 0))
    time_t = jnp.transpose(pv_time_utc_fourier, (1, 2, 0))
    time_sl = lax.dynamic_index_in_dim(time_t, t, 0, keepdims=False)
    az_sl = lax.dynamic_index_in_dim(jnp.transpose(solar_azimuth, (1, 0)),
                                     t, 0, keepdims=True)
    el_sl = lax.dynamic_index_in_dim(jnp.transpose(solar_elevation, (1, 0)),
                                     t, 0, keepdims=True)
    bc = jnp.concatenate(
        [time_sl, az_sl, el_sl, jnp.zeros((16 - F - 2, B), jnp.float32)], axis=0)
    out_t = _sc_call(y_t, x_t, sh_t, bc, idx_t, tab_t)
    return jnp.transpose(out_t, (2, 1, 0))


# speedup vs baseline: 6.0130x; 1.0071x over previous
import jax
import jax.numpy as jnp
from jax import lax
from jax.experimental import pallas as pl
from jax.experimental.pallas import tpu as pltpu
from jax.experimental.pallas import tpu_sc as plsc

B, P, F = 1024, 200, 8
V, D = 100000, 64
W = 2 * F + 1 + F + 2 + D
NC, NS = 2, 16
NW = NC * NS
PB = P // 8
H = B // 2


def _sc_body(y_hbm, x_hbm, sh_hbm, bc_hbm, idx_hbm, tab_hbm, out_hbm,
             row_v, ia_v, ib_v, oa_v, ob_v, sa_v, sb_v, bcr_v,
             sem_ia, sem_ib, sem_oa, sem_ob, sem_da, sem_db):
    sid = lax.axis_index("s")
    wid = sid * NC + lax.axis_index("c")

    @pl.when(wid < PB)
    def _dense():
        p0 = pl.multiple_of(wid * 8, 8)

        def src(t):
            kind, f, b0 = t
            if kind == 0:
                return y_hbm.at[pl.ds(p0, 8), f, pl.ds(b0, H)]
            if kind == 1:
                return x_hbm.at[pl.ds(p0, 8), f, pl.ds(b0, H)]
            return sh_hbm.at[pl.ds(p0, 8), pl.ds(b0, H)]

        def dst(t):
            kind, f, b0 = t
            fo = f if kind == 0 else (F + f if kind == 1 else 2 * F)
            return out_hbm.at[fo, pl.ds(p0, 8), pl.ds(b0, H)]

        transfers = [(kind, f, b0)
                     for kind in (0, 1) for f in range(F) for b0 in (0, H)]
        transfers += [(2, 0, 0), (2, 0, H)]
        bufs = [(sa_v, sem_da), (sb_v, sem_db)]
        handles = [None, None]
        handles[0] = pltpu.async_copy(src(transfers[0]), sa_v, sem_da)
        for t, tr in enumerate(transfers):
            s = t % 2
            if t + 1 < len(transfers):
                nb, nsem = bufs[(t + 1) % 2]
                handles[(t + 1) % 2] = pltpu.async_copy(
                    src(transfers[t + 1]), nb, nsem)
            handles[s].wait()
            pltpu.sync_copy(bufs[s][0], dst(tr))

    @pl.when(wid >= NW - 10)
    def _bcast():
        j = wid - (NW - 10)
        fo = 2 * F + 1 + j
        pltpu.sync_copy(bc_hbm.at[j, :], bcr_v)

        for half, st in ((0, sa_v), (H, sb_v)):
            def rep_body(c, carry, half=half, st=st):
                cl = pl.multiple_of(c * 16, 16)
                cg = pl.multiple_of(half + c * 16, 16)
                pat = bcr_v[pl.ds(cg, 16)]
                for r in range(8):
                    st[r, pl.ds(cl, 16)] = pat
                return carry

            lax.fori_loop(0, H // 16, rep_body, 0, unroll=4)

        def fire_body(pb, carry):
            p0 = pl.multiple_of(pb * 8, 8)
            pltpu.async_copy(sa_v, out_hbm.at[fo, pl.ds(p0, 8), pl.ds(0, H)],
                             sem_da)
            pltpu.async_copy(sb_v, out_hbm.at[fo, pl.ds(p0, 8), pl.ds(H, H)],
                             sem_db)
            return carry

        lax.fori_loop(0, PB, fire_body, 0)

        def drain_body(pb, carry):
            p0 = pl.multiple_of(pb * 8, 8)
            pltpu.make_async_copy(
                sa_v, out_hbm.at[fo, pl.ds(p0, 8), pl.ds(0, H)], sem_da).wait()
            pltpu.make_async_copy(
                sb_v, out_hbm.at[fo, pl.ds(p0, 8), pl.ds(H, H)], sem_db).wait()
            return carry

        lax.fori_loop(0, PB, drain_body, 0)

    def gather(idx_buf, os_buf):
        def g_body(c, carry):
            c16 = pl.multiple_of(c * 16, 16)
            ivs = [idx_buf[r, pl.ds(c16, 16)] for r in range(8)]
            gs = [plsc.load_gather(row_v, [iv]) for iv in ivs]
            for r in range(8):
                os_buf[r, pl.ds(c16, 16)] = gs[r]
            return carry

        lax.fori_loop(0, H // 16, g_body, 0, unroll=4)

    def idx_src(p0, b0):
        return idx_hbm.at[pl.ds(p0, 8), pl.ds(b0, H)]

    for t in range(2):
        k = wid * 2 + t
        fo = W - D + k
        pltpu.sync_copy(tab_hbm.at[k, :], row_v)

        def out_dst(p0, b0, fo=fo):
            return out_hbm.at[fo, pl.ds(p0, 8), pl.ds(b0, H)]

        pltpu.async_copy(idx_src(0, 0), ia_v, sem_ia)
        pltpu.async_copy(idx_src(0, H), ib_v, sem_ib)
        pltpu.make_async_copy(idx_src(0, 0), ia_v, sem_ia).wait()
        gather(ia_v, oa_v)
        pltpu.async_copy(oa_v, out_dst(0, 0), sem_oa)
        pltpu.async_copy(idx_src(8, 0), ia_v, sem_ia)
        pltpu.make_async_copy(idx_src(0, H), ib_v, sem_ib).wait()
        gather(ib_v, ob_v)
        pltpu.async_copy(ob_v, out_dst(0, H), sem_ob)
        pltpu.async_copy(idx_src(8, H), ib_v, sem_ib)

        def row_body(i, carry):
            p0 = pl.multiple_of(i * 8, 8)
            pn = pl.multiple_of(i * 8 + 8, 8)
            pltpu.make_async_copy(idx_src(p0, 0), ia_v, sem_ia).wait()
            pltpu.make_async_copy(oa_v, out_dst(p0, 0), sem_oa).wait()
            gather(ia_v, oa_v)
            pltpu.async_copy(oa_v, out_dst(p0, 0), sem_oa)
            pltpu.async_copy(idx_src(pn, 0), ia_v, sem_ia)
            pltpu.make_async_copy(idx_src(p0, H), ib_v, sem_ib).wait()
            pltpu.make_async_copy(ob_v, out_dst(p0, H), sem_ob).wait()
            gather(ib_v, ob_v)
            pltpu.async_copy(ob_v, out_dst(p0, H), sem_ob)
            pltpu.async_copy(idx_src(pn, H), ib_v, sem_ib)
            return carry

        lax.fori_loop(1, PB - 1, row_body, 0)

        pl_last = pl.multiple_of((PB - 1) * 8, 8)
        pltpu.make_async_copy(idx_src(pl_last, 0), ia_v, sem_ia).wait()
        pltpu.make_async_copy(oa_v, out_dst(pl_last, 0), sem_oa).wait()
        gather(ia_v, oa_v)
        pltpu.async_copy(oa_v, out_dst(pl_last, 0), sem_oa)
        pltpu.make_async_copy(idx_src(pl_last, H), ib_v, sem_ib).wait()
        pltpu.make_async_copy(ob_v, out_dst(pl_last, H), sem_ob).wait()
        gather(ib_v, ob_v)
        pltpu.async_copy(ob_v, out_dst(pl_last, H), sem_ob)
        pltpu.make_async_copy(oa_v, out_dst(pl_last, 0), sem_oa).wait()
        pltpu.make_async_copy(ob_v, out_dst(pl_last, H), sem_ob).wait()


_sc_call = pl.kernel(
    _sc_body,
    out_type=jax.ShapeDtypeStruct((W, P, B), jnp.float32),
    mesh=plsc.VectorSubcoreMesh(core_axis_name="c", subcore_axis_name="s"),
    scratch_types=[
        pltpu.VMEM((V,), jnp.float32),
        pltpu.VMEM((8, H), jnp.int32),
        pltpu.VMEM((8, H), jnp.int32),
        pltpu.VMEM((8, H), jnp.float32),
        pltpu.VMEM((8, H), jnp.float32),
        pltpu.VMEM((8, H), jnp.float32),
        pltpu.VMEM((8, H), jnp.float32),
        pltpu.VMEM((B,), jnp.float32),
        pltpu.SemaphoreType.DMA,
        pltpu.SemaphoreType.DMA,
        pltpu.SemaphoreType.DMA,
        pltpu.SemaphoreType.DMA,
        pltpu.SemaphoreType.DMA,
        pltpu.SemaphoreType.DMA,
    ],
    compiler_params=pltpu.CompilerParams(needs_layout_passes=False),
)


def kernel(pv_y_osgb_fourier, pv_x_osgb_fourier, pv_system_row_number, pv_x_osgb,
           pv_surface_height, pv_time_utc_fourier, solar_azimuth, solar_elevation,
           embedding_table, start_idx=0):
    t = 12 + start_idx
    y_t = jnp.transpose(pv_y_osgb_fourier, (1, 2, 0))
    x_t = jnp.transpose(pv_x_osgb_fourier, (1, 2, 0))
    sh_t = jnp.transpose(pv_surface_height, (1, 0))
    idx_t = jnp.transpose(pv_system_row_number.astype(jnp.int32), (1, 0))
    tab_t = jnp.transpose(embedding_table, (1, 0))
    time_t = jnp.transpose(pv_time_utc_fourier, (1, 2, 0))
    time_sl = lax.dynamic_index_in_dim(time_t, t, 0, keepdims=False)
    az_sl = lax.dynamic_index_in_dim(jnp.transpose(solar_azimuth, (1, 0)),
                                     t, 0, keepdims=True)
    el_sl = lax.dynamic_index_in_dim(jnp.transpose(solar_elevation, (1, 0)),
                                     t, 0, keepdims=True)
    bc = jnp.concatenate(
        [time_sl, az_sl, el_sl, jnp.zeros((16 - F - 2, B), jnp.float32)], axis=0)
    out_t = _sc_call(y_t, x_t, sh_t, bc, idx_t, tab_t)
    return jnp.transpose(out_t, (2, 1, 0))


# speedup vs baseline: 6.0182x; 1.0009x over previous
import jax
import jax.numpy as jnp
from jax import lax
from jax.experimental import pallas as pl
from jax.experimental.pallas import tpu as pltpu
from jax.experimental.pallas import tpu_sc as plsc

B, P, F = 1024, 200, 8
V, D = 100000, 64
W = 2 * F + 1 + F + 2 + D
NC, NS = 2, 16
NW = NC * NS
PB = P // 8
H = B // 2


def _sc_body(y_hbm, x_hbm, sh_hbm, bc_hbm, idx_hbm, tab_hbm, out_hbm,
             row_v, ia_v, ib_v, oa_v, ob_v, sa_v, sb_v, bcr_v, spidx,
             sem_ia, sem_ib, sem_oa, sem_ob, sem_da, sem_db, sem_sp):
    sid = lax.axis_index("s")
    wid = sid * NC + lax.axis_index("c")

    @pl.when(wid < PB)
    def _dense():
        p0 = pl.multiple_of(wid * 8, 8)

        def src(t):
            kind, f, b0 = t
            if kind == 0:
                return y_hbm.at[pl.ds(p0, 8), f, pl.ds(b0, H)]
            if kind == 1:
                return x_hbm.at[pl.ds(p0, 8), f, pl.ds(b0, H)]
            return sh_hbm.at[pl.ds(p0, 8), pl.ds(b0, H)]

        def dst(t):
            kind, f, b0 = t
            fo = f if kind == 0 else (F + f if kind == 1 else 2 * F)
            return out_hbm.at[fo, pl.ds(p0, 8), pl.ds(b0, H)]

        transfers = [(kind, f, b0)
                     for kind in (0, 1) for f in range(F) for b0 in (0, H)]
        transfers += [(2, 0, 0), (2, 0, H)]
        bufs = [(sa_v, sem_da), (sb_v, sem_db)]
        handles = [None, None]
        handles[0] = pltpu.async_copy(src(transfers[0]), sa_v, sem_da)
        for t, tr in enumerate(transfers):
            s = t % 2
            if t + 1 < len(transfers):
                nb, nsem = bufs[(t + 1) % 2]
                handles[(t + 1) % 2] = pltpu.async_copy(
                    src(transfers[t + 1]), nb, nsem)
            handles[s].wait()
            pltpu.sync_copy(bufs[s][0], dst(tr))

    @pl.when(wid >= NW - 10)
    def _bcast():
        j = wid - (NW - 10)
        fo = 2 * F + 1 + j
        pltpu.sync_copy(bc_hbm.at[j, :], bcr_v)

        for half, st in ((0, sa_v), (H, sb_v)):
            def rep_body(c, carry, half=half, st=st):
                cl = pl.multiple_of(c * 16, 16)
                cg = pl.multiple_of(half + c * 16, 16)
                pat = bcr_v[pl.ds(cg, 16)]
                for r in range(8):
                    st[r, pl.ds(cl, 16)] = pat
                return carry

            lax.fori_loop(0, H // 16, rep_body, 0, unroll=4)

        def fire_body(pb, carry):
            p0 = pl.multiple_of(pb * 8, 8)
            pltpu.async_copy(sa_v, out_hbm.at[fo, pl.ds(p0, 8), pl.ds(0, H)],
                             sem_da)
            pltpu.async_copy(sb_v, out_hbm.at[fo, pl.ds(p0, 8), pl.ds(H, H)],
                             sem_db)
            return carry

        lax.fori_loop(0, PB, fire_body, 0)

        def drain_body(pb, carry):
            p0 = pl.multiple_of(pb * 8, 8)
            pltpu.make_async_copy(
                sa_v, out_hbm.at[fo, pl.ds(p0, 8), pl.ds(0, H)], sem_da).wait()
            pltpu.make_async_copy(
                sb_v, out_hbm.at[fo, pl.ds(p0, 8), pl.ds(H, H)], sem_db).wait()
            return carry

        lax.fori_loop(0, PB, drain_body, 0)

    def gather(idx_buf, os_buf):
        def g_body(c, carry):
            c16 = pl.multiple_of(c * 16, 16)
            ivs = [idx_buf[r, pl.ds(c16, 16)] for r in range(8)]
            gs = [plsc.load_gather(row_v, [iv]) for iv in ivs]
            for r in range(8):
                os_buf[r, pl.ds(c16, 16)] = gs[r]
            return carry

        lax.fori_loop(0, H // 16, g_body, 0, unroll=4)

    is_loader = sid == 0

    def sp_src(i):
        p0 = pl.multiple_of(i * 8, 8)
        return idx_hbm.at[pl.ds(p0, 8), :]

    for t in range(2):
        k = wid * 2 + t
        fo = W - D + k
        pltpu.sync_copy(tab_hbm.at[k, :], row_v)

        def out_dst(p0, b0, fo=fo):
            return out_hbm.at[fo, pl.ds(p0, 8), pl.ds(b0, H)]

        plsc.subcore_barrier()

        @pl.when(is_loader)
        def _prefire():
            pltpu.async_copy(sp_src(0), spidx.at[0], sem_sp)

        def row_body(i, carry):
            p0 = pl.multiple_of(i * 8, 8)
            slot = lax.rem(i, 2)

            @pl.when(is_loader)
            def _wait_sp():
                pltpu.make_async_copy(sp_src(i), spidx.at[slot], sem_sp).wait()

            plsc.subcore_barrier()

            @pl.when(is_loader & (i < PB - 1))
            def _fire_next():
                pltpu.async_copy(sp_src(i + 1), spidx.at[1 - slot], sem_sp)

            pltpu.sync_copy(spidx.at[slot, :, pl.ds(0, H)], ia_v)

            @pl.when(i > 0)
            def _wait_oa():
                pltpu.make_async_copy(oa_v, out_dst(p0, 0), sem_oa).wait()

            gather(ia_v, oa_v)
            pltpu.async_copy(oa_v, out_dst(p0, 0), sem_oa)
            pltpu.sync_copy(spidx.at[slot, :, pl.ds(H, H)], ib_v)

            @pl.when(i > 0)
            def _wait_ob():
                pltpu.make_async_copy(ob_v, out_dst(p0, H), sem_ob).wait()

            gather(ib_v, ob_v)
            pltpu.async_copy(ob_v, out_dst(p0, H), sem_ob)
            return carry

        lax.fori_loop(0, PB, row_body, 0)
        pltpu.make_async_copy(oa_v, out_dst(0, 0), sem_oa).wait()
        pltpu.make_async_copy(ob_v, out_dst(0, H), sem_ob).wait()


_sc_call = pl.kernel(
    _sc_body,
    out_type=jax.ShapeDtypeStruct((W, P, B), jnp.float32),
    mesh=plsc.VectorSubcoreMesh(core_axis_name="c", subcore_axis_name="s"),
    scratch_types=[
        pltpu.VMEM((V,), jnp.float32),
        pltpu.VMEM((8, H), jnp.int32),
        pltpu.VMEM((8, H), jnp.int32),
        pltpu.VMEM((8, H), jnp.float32),
        pltpu.VMEM((8, H), jnp.float32),
        pltpu.VMEM((8, H), jnp.float32),
        pltpu.VMEM((8, H), jnp.float32),
        pltpu.VMEM((B,), jnp.float32),
        pltpu.VMEM_SHARED((2, 8, B), jnp.int32),
        pltpu.SemaphoreType.DMA,
        pltpu.SemaphoreType.DMA,
        pltpu.SemaphoreType.DMA,
        pltpu.SemaphoreType.DMA,
        pltpu.SemaphoreType.DMA,
        pltpu.SemaphoreType.DMA,
        pltpu.SemaphoreType.DMA,
    ],
    compiler_params=pltpu.CompilerParams(needs_layout_passes=False),
)


def kernel(pv_y_osgb_fourier, pv_x_osgb_fourier, pv_system_row_number, pv_x_osgb,
           pv_surface_height, pv_time_utc_fourier, solar_azimuth, solar_elevation,
           embedding_table, start_idx=0):
    t = 12 + start_idx
    y_t = jnp.transpose(pv_y_osgb_fourier, (1, 2, 0))
    x_t = jnp.transpose(pv_x_osgb_fourier, (1, 2, 0))
    sh_t = jnp.transpose(pv_surface_height, (1, 0))
    idx_t = jnp.transpose(pv_system_row_number.astype(jnp.int32), (1, 0))
    tab_t = jnp.transpose(embedding_table, (1, 0))
    time_t = jnp.transpose(pv_time_utc_fourier, (1, 2, 0))
    time_sl = lax.dynamic_index_in_dim(time_t, t, 0, keepdims=False)
    az_sl = lax.dynamic_index_in_dim(jnp.transpose(solar_azimuth, (1, 0)),
                                     t, 0, keepdims=True)
    el_sl = lax.dynamic_index_in_dim(jnp.transpose(solar_elevation, (1, 0)),
                                     t, 0, keepdims=True)
    bc = jnp.concatenate(
        [time_sl, az_sl, el_sl, jnp.zeros((16 - F - 2, B), jnp.float32)], axis=0)
    out_t = _sc_call(y_t, x_t, sh_t, bc, idx_t, tab_t)
    return jnp.transpose(out_t, (2, 1, 0))


# speedup vs baseline: 6.1897x; 1.0285x over previous
import jax
import jax.numpy as jnp
from jax import lax
from jax.experimental import pallas as pl
from jax.experimental.pallas import tpu as pltpu
from jax.experimental.pallas import tpu_sc as plsc

B, P, F = 1024, 200, 8
V, D = 100000, 64
W = 2 * F + 1 + F + 2 + D
NC, NS = 2, 16
NW = NC * NS
PB = P // 8
H = B // 2
NSLOT = 6
GRP = 5


def _sc_body(y_hbm, x_hbm, sh_hbm, bc_hbm, idx_hbm, tab_hbm, out_hbm,
             row_v, iab_v, oa_v, ob_v, sa_v, sb_v, bcr_v, spidx,
             sem_wa, sem_wb, sem_oa, sem_ob, sem_da, sem_db, sem_sp):
    sid = lax.axis_index("s")
    wid = sid * NC + lax.axis_index("c")

    @pl.when(wid < PB)
    def _dense():
        p0 = pl.multiple_of(wid * 8, 8)

        def src(t):
            kind, f, b0 = t
            if kind == 0:
                return y_hbm.at[pl.ds(p0, 8), f, pl.ds(b0, H)]
            if kind == 1:
                return x_hbm.at[pl.ds(p0, 8), f, pl.ds(b0, H)]
            return sh_hbm.at[pl.ds(p0, 8), pl.ds(b0, H)]

        def dst(t):
            kind, f, b0 = t
            fo = f if kind == 0 else (F + f if kind == 1 else 2 * F)
            return out_hbm.at[fo, pl.ds(p0, 8), pl.ds(b0, H)]

        transfers = [(kind, f, b0)
                     for kind in (0, 1) for f in range(F) for b0 in (0, H)]
        transfers += [(2, 0, 0), (2, 0, H)]
        n = len(transfers)
        rbufs = [(sa_v, sem_da), (sb_v, sem_db)]
        wsems = [sem_wa, sem_wb]
        hr = [None, None]
        hw = [None, None]
        hr[0] = pltpu.async_copy(src(transfers[0]), sa_v, sem_da)
        for t, tr in enumerate(transfers):
            b = t % 2
            if t + 1 < n:
                nb = (t + 1) % 2
                if hw[nb] is not None:
                    hw[nb].wait()
                hr[nb] = pltpu.async_copy(src(transfers[t + 1]),
                                          rbufs[nb][0], rbufs[nb][1])
            hr[b].wait()
            hw[b] = pltpu.async_copy(rbufs[b][0], dst(tr), wsems[b])
        hw[0].wait()
        hw[1].wait()

    @pl.when(wid >= NW - 10)
    def _bcast():
        j = wid - (NW - 10)
        fo = 2 * F + 1 + j
        pltpu.sync_copy(bc_hbm.at[j, :], bcr_v)

        for half, st in ((0, sa_v), (H, sb_v)):
            def rep_body(c, carry, half=half, st=st):
                cl = pl.multiple_of(c * 16, 16)
                cg = pl.multiple_of(half + c * 16, 16)
                pat = bcr_v[pl.ds(cg, 16)]
                for r in range(8):
                    st[r, pl.ds(cl, 16)] = pat
                return carry

            lax.fori_loop(0, H // 16, rep_body, 0, unroll=4)

        def fire_body(pb, carry):
            p0 = pl.multiple_of(pb * 8, 8)
            pltpu.async_copy(sa_v, out_hbm.at[fo, pl.ds(p0, 8), pl.ds(0, H)],
                             sem_da)
            pltpu.async_copy(sb_v, out_hbm.at[fo, pl.ds(p0, 8), pl.ds(H, H)],
                             sem_db)
            return carry

        lax.fori_loop(0, PB, fire_body, 0)

        def drain_body(pb, carry):
            p0 = pl.multiple_of(pb * 8, 8)
            pltpu.make_async_copy(
                sa_v, out_hbm.at[fo, pl.ds(p0, 8), pl.ds(0, H)], sem_da).wait()
            pltpu.make_async_copy(
                sb_v, out_hbm.at[fo, pl.ds(p0, 8), pl.ds(H, H)], sem_db).wait()
            return carry

        lax.fori_loop(0, PB, drain_body, 0)

    def gather(base, os_buf):
        def g_body(c, carry):
            cl = pl.multiple_of(c * 16, 16)
            cg = pl.multiple_of(base + c * 16, 16)
            ivs = [iab_v[r, pl.ds(cg, 16)] for r in range(8)]
            gs = [plsc.load_gather(row_v, [iv]) for iv in ivs]
            for r in range(8):
                os_buf[r, pl.ds(cl, 16)] = gs[r]
            return carry

        lax.fori_loop(0, H // 16, g_body, 0, unroll=4)

    is_loader = sid == 0

    def sp_src(i):
        p0 = pl.multiple_of(i * 8, 8)
        return idx_hbm.at[pl.ds(p0, 8), :]

    def sp_slot(i):
        return spidx.at[lax.rem(i, NSLOT)]

    for t in range(2):
        k = wid * 2 + t
        fo = W - D + k
        pltpu.sync_copy(tab_hbm.at[k, :], row_v)

        def out_dst(p0, b0, fo=fo):
            return out_hbm.at[fo, pl.ds(p0, 8), pl.ds(b0, H)]

        plsc.subcore_barrier()

        @pl.when(is_loader)
        def _prefire():
            for i in range(GRP):
                pltpu.async_copy(sp_src(i), spidx.at[i], sem_sp)

        def grp_body(g, carry):
            @pl.when(is_loader)
            def _wait_sp():
                for j in range(GRP):
                    i = g * GRP + j
                    pltpu.make_async_copy(sp_src(i), sp_slot(i), sem_sp).wait()

            plsc.subcore_barrier()

            @pl.when(is_loader & (g < PB // GRP - 1))
            def _fire_next():
                for j in range(GRP):
                    i = (g + 1) * GRP + j
                    pltpu.async_copy(sp_src(i), sp_slot(i), sem_sp)

            for j in range(GRP):
                i = g * GRP + j
                p0 = pl.multiple_of(i * 8, 8)
                pltpu.sync_copy(sp_slot(i), iab_v)
                if j == 0:
                    @pl.when(g > 0)
                    def _wait_oa():
                        pltpu.make_async_copy(
                            oa_v, out_dst(p0, 0), sem_oa).wait()
                else:
                    pltpu.make_async_copy(oa_v, out_dst(p0, 0), sem_oa).wait()
                gather(0, oa_v)
                pltpu.async_copy(oa_v, out_dst(p0, 0), sem_oa)
                if j == 0:
                    @pl.when(g > 0)
                    def _wait_ob():
                        pltpu.make_async_copy(
                            ob_v, out_dst(p0, H), sem_ob).wait()
                else:
                    pltpu.make_async_copy(ob_v, out_dst(p0, H), sem_ob).wait()
                gather(H, ob_v)
                pltpu.async_copy(ob_v, out_dst(p0, H), sem_ob)
            return carry

        lax.fori_loop(0, PB // GRP, grp_body, 0)
        pltpu.make_async_copy(oa_v, out_dst(0, 0), sem_oa).wait()
        pltpu.make_async_copy(ob_v, out_dst(0, H), sem_ob).wait()


_sc_call = pl.kernel(
    _sc_body,
    out_type=jax.ShapeDtypeStruct((W, P, B), jnp.float32),
    mesh=plsc.VectorSubcoreMesh(core_axis_name="c", subcore_axis_name="s"),
    scratch_types=[
        pltpu.VMEM((V,), jnp.float32),
        pltpu.VMEM((8, B), jnp.int32),
        pltpu.VMEM((8, H), jnp.float32),
        pltpu.VMEM((8, H), jnp.float32),
        pltpu.VMEM((8, H), jnp.float32),
        pltpu.VMEM((8, H), jnp.float32),
        pltpu.VMEM((B,), jnp.float32),
        pltpu.VMEM_SHARED((NSLOT, 8, B), jnp.int32),
        pltpu.SemaphoreType.DMA,
        pltpu.SemaphoreType.DMA,
        pltpu.SemaphoreType.DMA,
        pltpu.SemaphoreType.DMA,
        pltpu.SemaphoreType.DMA,
        pltpu.SemaphoreType.DMA,
        pltpu.SemaphoreType.DMA,
    ],
    compiler_params=pltpu.CompilerParams(needs_layout_passes=False),
)


def kernel(pv_y_osgb_fourier, pv_x_osgb_fourier, pv_system_row_number, pv_x_osgb,
           pv_surface_height, pv_time_utc_fourier, solar_azimuth, solar_elevation,
           embedding_table, start_idx=0):
    t = 12 + start_idx
    y_t = jnp.transpose(pv_y_osgb_fourier, (1, 2, 0))
    x_t = jnp.transpose(pv_x_osgb_fourier, (1, 2, 0))
    sh_t = jnp.transpose(pv_surface_height, (1, 0))
    idx_t = jnp.transpose(pv_system_row_number.astype(jnp.int32), (1, 0))
    tab_t = jnp.transpose(embedding_table, (1, 0))
    time_t = jnp.transpose(pv_time_utc_fourier, (1, 2, 0))
    time_sl = lax.dynamic_index_in_dim(time_t, t, 0, keepdims=False)
    az_sl = lax.dynamic_index_in_dim(jnp.transpose(solar_azimuth, (1, 0)),
                                     t, 0, keepdims=True)
    el_sl = lax.dynamic_index_in_dim(jnp.transpose(solar_elevation, (1, 0)),
                                     t, 0, keepdims=True)
    bc = jnp.concatenate(
        [time_sl, az_sl, el_sl, jnp.zeros((16 - F - 2, B), jnp.float32)], axis=0)
    out_t = _sc_call(y_t, x_t, sh_t, bc, idx_t, tab_t)
    return jnp.transpose(out_t, (2, 1, 0))


# speedup vs baseline: 6.2036x; 1.0022x over previous
import jax
import jax.numpy as jnp
from jax import lax
from jax.experimental import pallas as pl
from jax.experimental.pallas import tpu as pltpu
from jax.experimental.pallas import tpu_sc as plsc

B, P, F = 1024, 200, 8
V, D = 100000, 64
W = 2 * F + 1 + F + 2 + D
NC, NS = 2, 16
NW = NC * NS
PB = P // 8
H = B // 2
NSLOT = 10
GRP = 5


def _sc_body(y_hbm, x_hbm, sh_hbm, bc_hbm, idx_hbm, tab_hbm, out_hbm,
             row_v, iab_v, oa_v, ob_v, sa_v, sb_v, bcr_v, spidx,
             sem_wa, sem_wb, sem_oa, sem_ob, sem_da, sem_db, sem_sp):
    sid = lax.axis_index("s")
    wid = sid * NC + lax.axis_index("c")

    @pl.when(wid < PB)
    def _dense():
        p0 = pl.multiple_of(wid * 8, 8)

        def src(t):
            kind, f, b0 = t
            if kind == 0:
                return y_hbm.at[pl.ds(p0, 8), f, pl.ds(b0, H)]
            if kind == 1:
                return x_hbm.at[pl.ds(p0, 8), f, pl.ds(b0, H)]
            return sh_hbm.at[pl.ds(p0, 8), pl.ds(b0, H)]

        def dst(t):
            kind, f, b0 = t
            fo = f if kind == 0 else (F + f if kind == 1 else 2 * F)
            return out_hbm.at[fo, pl.ds(p0, 8), pl.ds(b0, H)]

        transfers = [(kind, f, b0)
                     for kind in (0, 1) for f in range(F) for b0 in (0, H)]
        transfers += [(2, 0, 0), (2, 0, H)]
        n = len(transfers)
        rbufs = [(sa_v, sem_da), (sb_v, sem_db)]
        wsems = [sem_wa, sem_wb]
        hr = [None, None]
        hw = [None, None]
        hr[0] = pltpu.async_copy(src(transfers[0]), sa_v, sem_da)
        for t, tr in enumerate(transfers):
            b = t % 2
            if t + 1 < n:
                nb = (t + 1) % 2
                if hw[nb] is not None:
                    hw[nb].wait()
                hr[nb] = pltpu.async_copy(src(transfers[t + 1]),
                                          rbufs[nb][0], rbufs[nb][1])
            hr[b].wait()
            hw[b] = pltpu.async_copy(rbufs[b][0], dst(tr), wsems[b])
        hw[0].wait()
        hw[1].wait()

    @pl.when(wid >= NW - 10)
    def _bcast():
        j = wid - (NW - 10)
        fo = 2 * F + 1 + j
        pltpu.sync_copy(bc_hbm.at[j, :], bcr_v)

        for half, st in ((0, sa_v), (H, sb_v)):
            def rep_body(c, carry, half=half, st=st):
                cl = pl.multiple_of(c * 16, 16)
                cg = pl.multiple_of(half + c * 16, 16)
                pat = bcr_v[pl.ds(cg, 16)]
                for r in range(8):
                    st[r, pl.ds(cl, 16)] = pat
                return carry

            lax.fori_loop(0, H // 16, rep_body, 0, unroll=4)

        def fire_body(pb, carry):
            p0 = pl.multiple_of(pb * 8, 8)
            pltpu.async_copy(sa_v, out_hbm.at[fo, pl.ds(p0, 8), pl.ds(0, H)],
                             sem_da)
            pltpu.async_copy(sb_v, out_hbm.at[fo, pl.ds(p0, 8), pl.ds(H, H)],
                             sem_db)
            return carry

        lax.fori_loop(0, PB, fire_body, 0)

        def drain_body(pb, carry):
            p0 = pl.multiple_of(pb * 8, 8)
            pltpu.make_async_copy(
                sa_v, out_hbm.at[fo, pl.ds(p0, 8), pl.ds(0, H)], sem_da).wait()
            pltpu.make_async_copy(
                sb_v, out_hbm.at[fo, pl.ds(p0, 8), pl.ds(H, H)], sem_db).wait()
            return carry

        lax.fori_loop(0, PB, drain_body, 0)

    def gather(base, os_buf):
        def g_body(c, carry):
            cl = pl.multiple_of(c * 16, 16)
            cg = pl.multiple_of(base + c * 16, 16)
            ivs = [iab_v[r, pl.ds(cg, 16)] for r in range(8)]
            gs = [plsc.load_gather(row_v, [iv]) for iv in ivs]
            for r in range(8):
                os_buf[r, pl.ds(cl, 16)] = gs[r]
            return carry

        lax.fori_loop(0, H // 16, g_body, 0, unroll=4)

    is_loader = sid == 0

    def sp_src(i):
        p0 = pl.multiple_of(i * 8, 8)
        return idx_hbm.at[pl.ds(p0, 8), :]

    def sp_slot(i):
        return spidx.at[lax.rem(i, NSLOT)]

    for t in range(2):
        k = wid * 2 + t
        fo = W - D + k
        pltpu.sync_copy(tab_hbm.at[k, :], row_v)

        def out_dst(p0, b0, fo=fo):
            return out_hbm.at[fo, pl.ds(p0, 8), pl.ds(b0, H)]

        plsc.subcore_barrier()

        @pl.when(is_loader)
        def _prefire():
            for i in range(GRP):
                pltpu.async_copy(sp_src(i), spidx.at[i], sem_sp)

        def grp_body(g, carry):
            @pl.when(is_loader)
            def _wait_sp():
                for j in range(GRP):
                    i = g * GRP + j
                    pltpu.make_async_copy(sp_src(i), sp_slot(i), sem_sp).wait()

            plsc.subcore_barrier()

            @pl.when(is_loader & (g < PB // GRP - 1))
            def _fire_next():
                for j in range(GRP):
                    i = (g + 1) * GRP + j
                    pltpu.async_copy(sp_src(i), sp_slot(i), sem_sp)

            for j in range(GRP):
                i = g * GRP + j
                p0 = pl.multiple_of(i * 8, 8)
                pltpu.sync_copy(sp_slot(i), iab_v)
                if j == 0:
                    @pl.when(g > 0)
                    def _wait_oa():
                        pltpu.make_async_copy(
                            oa_v, out_dst(p0, 0), sem_oa).wait()
                else:
                    pltpu.make_async_copy(oa_v, out_dst(p0, 0), sem_oa).wait()
                gather(0, oa_v)
                pltpu.async_copy(oa_v, out_dst(p0, 0), sem_oa)
                if j == 0:
                    @pl.when(g > 0)
                    def _wait_ob():
                        pltpu.make_async_copy(
                            ob_v, out_dst(p0, H), sem_ob).wait()
                else:
                    pltpu.make_async_copy(ob_v, out_dst(p0, H), sem_ob).wait()
                gather(H, ob_v)
                pltpu.async_copy(ob_v, out_dst(p0, H), sem_ob)
            return carry

        lax.fori_loop(0, PB // GRP, grp_body, 0)
        pltpu.make_async_copy(oa_v, out_dst(0, 0), sem_oa).wait()
        pltpu.make_async_copy(ob_v, out_dst(0, H), sem_ob).wait()


_sc_call = pl.kernel(
    _sc_body,
    out_type=jax.ShapeDtypeStruct((W, P, B), jnp.float32),
    mesh=plsc.VectorSubcoreMesh(core_axis_name="c", subcore_axis_name="s"),
    scratch_types=[
        pltpu.VMEM((V,), jnp.float32),
        pltpu.VMEM((8, B), jnp.int32),
        pltpu.VMEM((8, H), jnp.float32),
        pltpu.VMEM((8, H), jnp.float32),
        pltpu.VMEM((8, H), jnp.float32),
        pltpu.VMEM((8, H), jnp.float32),
        pltpu.VMEM((B,), jnp.float32),
        pltpu.VMEM_SHARED((NSLOT, 8, B), jnp.int32),
        pltpu.SemaphoreType.DMA,
        pltpu.SemaphoreType.DMA,
        pltpu.SemaphoreType.DMA,
        pltpu.SemaphoreType.DMA,
        pltpu.SemaphoreType.DMA,
        pltpu.SemaphoreType.DMA,
        pltpu.SemaphoreType.DMA,
    ],
    compiler_params=pltpu.CompilerParams(needs_layout_passes=False),
)


def kernel(pv_y_osgb_fourier, pv_x_osgb_fourier, pv_system_row_number, pv_x_osgb,
           pv_surface_height, pv_time_utc_fourier, solar_azimuth, solar_elevation,
           embedding_table, start_idx=0):
    t = 12 + start_idx
    y_t = jnp.transpose(pv_y_osgb_fourier, (1, 2, 0))
    x_t = jnp.transpose(pv_x_osgb_fourier, (1, 2, 0))
    sh_t = jnp.transpose(pv_surface_height, (1, 0))
    idx_t = jnp.transpose(pv_system_row_number.astype(jnp.int32), (1, 0))
    tab_t = jnp.transpose(embedding_table, (1, 0))
    time_t = jnp.transpose(pv_time_utc_fourier, (1, 2, 0))
    time_sl = lax.dynamic_index_in_dim(time_t, t, 0, keepdims=False)
    az_sl = lax.dynamic_index_in_dim(jnp.transpose(solar_azimuth, (1, 0)),
                                     t, 0, keepdims=True)
    el_sl = lax.dynamic_index_in_dim(jnp.transpose(solar_elevation, (1, 0)),
                                     t, 0, keepdims=True)
    bc = jnp.concatenate(
        [time_sl, az_sl, el_sl, jnp.zeros((16 - F - 2, B), jnp.float32)], axis=0)
    out_t = _sc_call(y_t, x_t, sh_t, bc, idx_t, tab_t)
    return jnp.transpose(out_t, (2, 1, 0))


# speedup vs baseline: 7.1401x; 1.1510x over previous
import jax
import jax.numpy as jnp
from jax import lax
from jax.experimental import pallas as pl
from jax.experimental.pallas import tpu as pltpu
from jax.experimental.pallas import tpu_sc as plsc

B, P, F = 1024, 200, 8
V, D = 100000, 64
W = 2 * F + 1 + F + 2 + D
NC, NS = 2, 16
NW = NC * NS
PB = P // 8
H = B // 2
NSLOT = 10
GRP = 5


def _sc_body(y_hbm, x_hbm, sh_hbm, bc_hbm, idx_hbm, tab_hbm, out_hbm,
             row_v, iab_v, oa_v, ob_v, ia2_v, ib2_v, bcr_v, spidx,
             sem_wa, sem_wb, sem_oa, sem_ob, sem_da, sem_db, sem_sp, sem_row):
    sid = lax.axis_index("s")
    wid = sid * NC + lax.axis_index("c")

    pltpu.async_copy(tab_hbm.at[wid * 2, :], row_v, sem_row)

    @pl.when(wid < PB)
    def _dense():
        p0 = pl.multiple_of(wid * 8, 8)

        def src(t):
            kind, f, b0 = t
            if kind == 0:
                return y_hbm.at[pl.ds(p0, 8), f, pl.ds(b0, H)]
            if kind == 1:
                return x_hbm.at[pl.ds(p0, 8), f, pl.ds(b0, H)]
            return sh_hbm.at[pl.ds(p0, 8), pl.ds(b0, H)]

        def dst(t):
            kind, f, b0 = t
            fo = f if kind == 0 else (F + f if kind == 1 else 2 * F)
            return out_hbm.at[fo, pl.ds(p0, 8), pl.ds(b0, H)]

        transfers = [(kind, f, b0)
                     for kind in (0, 1) for f in range(F) for b0 in (0, H)]
        transfers += [(2, 0, 0), (2, 0, H)]
        n = len(transfers)
        rbufs = [(oa_v, sem_da), (ob_v, sem_db)]
        wsems = [sem_wa, sem_wb]
        hr = [None, None]
        hw = [None, None]
        hr[0] = pltpu.async_copy(src(transfers[0]), oa_v, sem_da)
        for t, tr in enumerate(transfers):
            b = t % 2
            if t + 1 < n:
                nb = (t + 1) % 2
                if hw[nb] is not None:
                    hw[nb].wait()
                hr[nb] = pltpu.async_copy(src(transfers[t + 1]),
                                          rbufs[nb][0], rbufs[nb][1])
            hr[b].wait()
            hw[b] = pltpu.async_copy(rbufs[b][0], dst(tr), wsems[b])
        hw[0].wait()
        hw[1].wait()

    @pl.when(wid >= NW - 10)
    def _bcast():
        j = wid - (NW - 10)
        fo = 2 * F + 1 + j
        pltpu.sync_copy(bc_hbm.at[j, :], bcr_v)

        for half, st in ((0, oa_v), (H, ob_v)):
            def rep_body(c, carry, half=half, st=st):
                cl = pl.multiple_of(c * 16, 16)
                cg = pl.multiple_of(half + c * 16, 16)
                pat = bcr_v[pl.ds(cg, 16)]
                for r in range(8):
                    st[r, pl.ds(cl, 16)] = pat
                return carry

            lax.fori_loop(0, H // 16, rep_body, 0, unroll=4)

        def fire_body(pb, carry):
            p0 = pl.multiple_of(pb * 8, 8)
            pltpu.async_copy(oa_v, out_hbm.at[fo, pl.ds(p0, 8), pl.ds(0, H)],
                             sem_da)
            pltpu.async_copy(ob_v, out_hbm.at[fo, pl.ds(p0, 8), pl.ds(H, H)],
                             sem_db)
            return carry

        lax.fori_loop(0, PB, fire_body, 0)

        def drain_body(pb, carry):
            p0 = pl.multiple_of(pb * 8, 8)
            pltpu.make_async_copy(
                oa_v, out_hbm.at[fo, pl.ds(p0, 8), pl.ds(0, H)], sem_da).wait()
            pltpu.make_async_copy(
                ob_v, out_hbm.at[fo, pl.ds(p0, 8), pl.ds(H, H)], sem_db).wait()
            return carry

        lax.fori_loop(0, PB, drain_body, 0)

    def gather(src_ref, base, os_buf, cast):
        def g_body(c, carry):
            cl = pl.multiple_of(c * 16, 16)
            cg = pl.multiple_of(base + c * 16, 16)
            ivs = [src_ref[r, pl.ds(cg, 16)] for r in range(8)]
            if cast:
                ivs = [plsc.bitcast(iv, jnp.int32) for iv in ivs]
            gs = [plsc.load_gather(row_v, [iv]) for iv in ivs]
            for r in range(8):
                os_buf[r, pl.ds(cl, 16)] = gs[r]
            return carry

        lax.fori_loop(0, H // 16, g_body, 0, unroll=4)

    is_loader = sid == 0

    def sp_src(i):
        p0 = pl.multiple_of(i * 8, 8)
        return idx_hbm.at[pl.ds(p0, 8), :]

    def sp_slot(i):
        return spidx.at[lax.rem(i, NSLOT)]

    def sp_half(i, h):
        return spidx.at[lax.rem(i, NSLOT), :, pl.ds(h * H, H)]

    for t in range(2):
        k = wid * 2 + t
        fo = W - D + k
        pltpu.make_async_copy(tab_hbm.at[k, :], row_v, sem_row).wait()

        def out_dst(p0, b0, fo=fo):
            return out_hbm.at[fo, pl.ds(p0, 8), pl.ds(b0, H)]

        plsc.subcore_barrier()

        @pl.when(is_loader)
        def _prefire():
            for i in range(GRP):
                pltpu.async_copy(sp_src(i), spidx.at[i], sem_sp)

        def grp_body(g, carry):
            @pl.when(is_loader)
            def _wait_sp():
                for j in range(GRP):
                    i = g * GRP + j
                    pltpu.make_async_copy(sp_src(i), sp_slot(i), sem_sp).wait()

            plsc.subcore_barrier()

            @pl.when(is_loader & (g < PB // GRP - 1))
            def _fire_next():
                for j in range(GRP):
                    i = (g + 1) * GRP + j
                    pltpu.async_copy(sp_src(i), sp_slot(i), sem_sp)

            for j in range(GRP):
                i = g * GRP + j
                p0 = pl.multiple_of(i * 8, 8)
                if j == 0:
                    pltpu.sync_copy(sp_slot(i), iab_v)
                    pltpu.async_copy(sp_half(i + 1, 0), ia2_v, sem_da)
                    pltpu.async_copy(sp_half(i + 1, 1), ib2_v, sem_db)
                elif j % 2 == 1:
                    if j + 1 < GRP:
                        pltpu.async_copy(sp_slot(i + 1), iab_v, sem_wa)
                    pltpu.make_async_copy(sp_half(i, 0), ia2_v, sem_da).wait()
                    pltpu.make_async_copy(sp_half(i, 1), ib2_v, sem_db).wait()
                else:
                    if j + 1 < GRP:
                        pltpu.async_copy(sp_half(i + 1, 0), ia2_v, sem_da)
                        pltpu.async_copy(sp_half(i + 1, 1), ib2_v, sem_db)
                    pltpu.make_async_copy(sp_slot(i), iab_v, sem_wa).wait()
                use1 = (j % 2 == 1)
                if j == 0:
                    @pl.when(g > 0)
                    def _wait_oa():
                        pltpu.make_async_copy(
                            oa_v, out_dst(p0, 0), sem_oa).wait()
                else:
                    pltpu.make_async_copy(oa_v, out_dst(p0, 0), sem_oa).wait()
                if use1:
                    gather(ia2_v, 0, oa_v, False)
                else:
                    gather(iab_v, 0, oa_v, False)
                pltpu.async_copy(oa_v, out_dst(p0, 0), sem_oa)
                if j == 0:
                    @pl.when(g > 0)
                    def _wait_ob():
                        pltpu.make_async_copy(
                            ob_v, out_dst(p0, H), sem_ob).wait()
                else:
                    pltpu.make_async_copy(ob_v, out_dst(p0, H), sem_ob).wait()
                if use1:
                    gather(ib2_v, 0, ob_v, False)
                else:
                    gather(iab_v, H, ob_v, False)
                pltpu.async_copy(ob_v, out_dst(p0, H), sem_ob)
            return carry

        lax.fori_loop(0, PB // GRP, grp_body, 0)
        if t == 0:
            pltpu.async_copy(tab_hbm.at[wid * 2 + 1, :], row_v, sem_row)
        pltpu.make_async_copy(oa_v, out_dst(0, 0), sem_oa).wait()
        pltpu.make_async_copy(ob_v, out_dst(0, H), sem_ob).wait()


_sc_call = pl.kernel(
    _sc_body,
    out_type=jax.ShapeDtypeStruct((W, P, B), jnp.float32),
    mesh=plsc.VectorSubcoreMesh(core_axis_name="c", subcore_axis_name="s"),
    scratch_types=[
        pltpu.VMEM((V,), jnp.float32),
        pltpu.VMEM((8, B), jnp.int32),
        pltpu.VMEM((8, H), jnp.float32),
        pltpu.VMEM((8, H), jnp.float32),
        pltpu.VMEM((8, H), jnp.int32),
        pltpu.VMEM((8, H), jnp.int32),
        pltpu.VMEM((B,), jnp.float32),
        pltpu.VMEM_SHARED((NSLOT, 8, B), jnp.int32),
        pltpu.SemaphoreType.DMA,
        pltpu.SemaphoreType.DMA,
        pltpu.SemaphoreType.DMA,
        pltpu.SemaphoreType.DMA,
        pltpu.SemaphoreType.DMA,
        pltpu.SemaphoreType.DMA,
        pltpu.SemaphoreType.DMA,
        pltpu.SemaphoreType.DMA,
    ],
    compiler_params=pltpu.CompilerParams(needs_layout_passes=False),
)


def kernel(pv_y_osgb_fourier, pv_x_osgb_fourier, pv_system_row_number, pv_x_osgb,
           pv_surface_height, pv_time_utc_fourier, solar_azimuth, solar_elevation,
           embedding_table, start_idx=0):
    t = 12 + start_idx
    y_t = jnp.transpose(pv_y_osgb_fourier, (1, 2, 0))
    x_t = jnp.transpose(pv_x_osgb_fourier, (1, 2, 0))
    sh_t = jnp.transpose(pv_surface_height, (1, 0))
    idx_t = jnp.transpose(pv_system_row_number.astype(jnp.int32), (1, 0))
    tab_t = jnp.transpose(embedding_table, (1, 0))
    time_t = jnp.transpose(pv_time_utc_fourier, (1, 2, 0))
    time_sl = lax.dynamic_index_in_dim(time_t, t, 0, keepdims=False)
    az_sl = lax.dynamic_index_in_dim(jnp.transpose(solar_azimuth, (1, 0)),
                                     t, 0, keepdims=True)
    el_sl = lax.dynamic_index_in_dim(jnp.transpose(solar_elevation, (1, 0)),
                                     t, 0, keepdims=True)
    bc = jnp.concatenate(
        [time_sl, az_sl, el_sl, jnp.zeros((16 - F - 2, B), jnp.float32)], axis=0)
    out_t = _sc_call(y_t, x_t, sh_t, bc, idx_t, tab_t)
    return jnp.transpose(out_t, (2, 1, 0))
